# baseline (speedup 1.0000x reference)
import functools

import jax
import jax.numpy as jnp
from jax import lax
from jax.experimental import pallas as pl
from jax.experimental.pallas import tpu as pltpu
from jax.experimental.pallas import tpu_sc as plsc

_BATCH = 16384
_HALF = _BATCH // 2
_ROWS = 1000
_D_IN = 64
_D_HID = 128
_D_OUT = 64
_BLK = 4096


def _make_sc_gather():
    info = plsc.get_sparse_core_info()
    nw = info.num_cores * info.num_subcores
    bpw = _BATCH // nw
    ch = 128
    nch = bpw // ch
    mesh = plsc.VectorSubcoreMesh(core_axis_name="c", subcore_axis_name="s")

    hpw = bpw // 2

    @functools.partial(
        pl.kernel,
        mesh=mesh,
        compiler_params=pltpu.CompilerParams(
            use_tc_tiling_on_sc=False, needs_layout_passes=False
        ),
        out_type=jax.ShapeDtypeStruct((_BATCH, _D_IN), jnp.float32),
        scratch_types=[
            pltpu.VMEM((hpw,), jnp.int32),
            pltpu.VMEM((hpw,), jnp.int32),
            pltpu.VMEM((bpw,), jnp.int32),
            pltpu.VMEM((bpw, _D_IN), jnp.float32),
            pltpu.SemaphoreType.DMA,
        ],
    )
    def gather(tbl_hbm, idx_hbm, out_hbm, idx_lo, idx_hi, idx_v, rows_v, sem):
        wid = lax.axis_index("s") * info.num_cores + lax.axis_index("c")
        base = wid * bpw
        pltpu.sync_copy(idx_hbm.at[pl.ds(wid * hpw, hpw)], idx_lo)
        pltpu.sync_copy(idx_hbm.at[pl.ds(_HALF + wid * hpw, hpw)], idx_hi)
        lanes2 = lax.iota(jnp.int32, 16) * 2
        for k in range(hpw // 16):
            pos = lanes2 + (32 * k)
            plsc.store_scatter(idx_v, [pos], idx_lo[pl.ds(k * 16, 16)])
            plsc.store_scatter(idx_v, [pos + 1], idx_hi[pl.ds(k * 16, 16)])
        copies = [
            pltpu.async_copy(
                tbl_hbm.at[idx_v.at[pl.ds(j * ch, ch)]],
                rows_v.at[pl.ds(j * ch, ch)],
                sem,
            )
            for j in range(nch)
        ]
        for c in copies:
            c.wait()
        pltpu.sync_copy(rows_v, out_hbm.at[pl.ds(base, bpw)])

    return gather


_sc_gather = _make_sc_gather()


def _mish(h):
    u = jnp.exp(h)
    num = u * (u + 2.0)
    return h * num / (num + 2.0)


def _mlp_body(xp_ref, w1_ref, b1_ref, w2t_ref, b2_ref, out_ref):
    h_idx = pl.program_id(1)
    x = jnp.where(h_idx == 0, xp_ref[:, :_D_IN], xp_ref[:, _D_IN:])
    h = _mish(
        jnp.dot(x, w1_ref[...], preferred_element_type=jnp.float32) + b1_ref[...]
    )
    out_ref[...] = (
        jax.lax.dot_general(
            w2t_ref[...], h, (((1,), (1,)), ((), ())),
            preferred_element_type=jnp.float32,
        )
        + b2_ref[...]
    )


def _pair_mlp_t(xp, W1, b1, W2t, b2):
    n_blk = _HALF // _BLK
    return pl.pallas_call(
        _mlp_body,
        grid=(n_blk, 2),
        in_specs=[
            pl.BlockSpec((_BLK, 2 * _D_IN), lambda j, h: (j, 0)),
            pl.BlockSpec((_D_IN, _D_HID), lambda j, h: (0, 0)),
            pl.BlockSpec((1, _D_HID), lambda j, h: (0, 0)),
            pl.BlockSpec((_D_OUT, _D_HID), lambda j, h: (0, 0)),
            pl.BlockSpec((_D_OUT, 1), lambda j, h: (0, 0)),
        ],
        out_specs=pl.BlockSpec((_D_OUT, _BLK), lambda j, h: (0, h * n_blk + j)),
        out_shape=jax.ShapeDtypeStruct((_D_OUT, _BATCH), jnp.float32),
    )(xp, W1, b1.reshape(1, -1), W2t, b2.reshape(-1, 1))


def kernel(diffusion_step, embedding, W1, b1, W2, b2):
    idx = diffusion_step.astype(jnp.int32)
    x = _sc_gather(embedding, idx)
    xp = x.reshape(_HALF, 2 * _D_IN)
    out_t = _pair_mlp_t(xp, W1, b1, W2.T, b2)
    return out_t.T


# speedup vs baseline: 1.0243x; 1.0243x over previous
import functools

import jax
import jax.numpy as jnp
from jax import lax
from jax.experimental import pallas as pl
from jax.experimental.pallas import tpu as pltpu
from jax.experimental.pallas import tpu_sc as plsc

_BATCH = 16384
_HALF = _BATCH // 2
_ROWS = 1000
_D_IN = 64
_D_HID = 128
_D_OUT = 64
_BLK = 4096


def _make_sc_gather():
    info = plsc.get_sparse_core_info()
    nw = info.num_cores * info.num_subcores
    bpw = _BATCH // nw
    ch = 128
    nch = bpw // ch
    mesh = plsc.VectorSubcoreMesh(core_axis_name="c", subcore_axis_name="s")

    hpw = bpw // 2

    @functools.partial(
        pl.kernel,
        mesh=mesh,
        compiler_params=pltpu.CompilerParams(
            use_tc_tiling_on_sc=False, needs_layout_passes=False
        ),
        out_type=jax.ShapeDtypeStruct((_BATCH, _D_IN), jnp.float32),
        scratch_types=[
            pltpu.VMEM((hpw,), jnp.int32),
            pltpu.VMEM((hpw,), jnp.int32),
            pltpu.VMEM((bpw,), jnp.int32),
            pltpu.VMEM((bpw, _D_IN), jnp.float32),
            pltpu.SemaphoreType.DMA,
        ],
    )
    def gather(tbl_hbm, idx_hbm, out_hbm, idx_lo, idx_hi, idx_v, rows_v, sem):
        wid = lax.axis_index("s") * info.num_cores + lax.axis_index("c")
        base = wid * bpw
        pltpu.sync_copy(idx_hbm.at[pl.ds(wid * hpw, hpw)], idx_lo)
        pltpu.sync_copy(idx_hbm.at[pl.ds(_HALF + wid * hpw, hpw)], idx_hi)
        lanes2 = lax.iota(jnp.int32, 16) * 2
        for k in range(hpw // 16):
            pos = lanes2 + (32 * k)
            plsc.store_scatter(idx_v, [pos], idx_lo[pl.ds(k * 16, 16)])
            plsc.store_scatter(idx_v, [pos + 1], idx_hi[pl.ds(k * 16, 16)])
        copies = [
            pltpu.async_copy(
                tbl_hbm.at[idx_v.at[pl.ds(j * ch, ch)]],
                rows_v.at[pl.ds(j * ch, ch)],
                sem,
            )
            for j in range(nch)
        ]
        for c in copies:
            c.wait()
        pltpu.sync_copy(rows_v, out_hbm.at[pl.ds(base, bpw)])

    return gather


_sc_gather = _make_sc_gather()


def _mish(h):
    u = jnp.exp(h)
    num = u * (u + 2.0)
    return h * num / (num + 2.0)


def _mlp_body(xp_ref, w1_ref, b1_ref, w2t_ref, b2_ref, out_ref):
    j = pl.program_id(0)
    w1 = w1_ref[...]
    b1 = b1_ref[...]
    w2t = w2t_ref[...]
    b2 = b2_ref[...]
    for half, col0 in ((0, 0), (1, _HALF)):
        x = xp_ref[:, half * _D_IN : (half + 1) * _D_IN]
        h = _mish(jnp.dot(x, w1, preferred_element_type=jnp.float32) + b1)
        o = (
            jax.lax.dot_general(
                w2t, h, (((1,), (1,)), ((), ())),
                preferred_element_type=jnp.float32,
            )
            + b2
        )
        out_ref[:, pl.ds(col0 + j * _BLK, _BLK)] = o


def _pair_mlp_t(xp, W1, b1, W2t, b2):
    n_blk = _HALF // _BLK
    return pl.pallas_call(
        _mlp_body,
        grid=(n_blk,),
        in_specs=[
            pl.BlockSpec((_BLK, 2 * _D_IN), lambda i: (i, 0)),
            pl.BlockSpec((_D_IN, _D_HID), lambda i: (0, 0)),
            pl.BlockSpec((1, _D_HID), lambda i: (0, 0)),
            pl.BlockSpec((_D_OUT, _D_HID), lambda i: (0, 0)),
            pl.BlockSpec((_D_OUT, 1), lambda i: (0, 0)),
        ],
        out_specs=pl.BlockSpec((_D_OUT, _BATCH), lambda i: (0, 0)),
        out_shape=jax.ShapeDtypeStruct((_D_OUT, _BATCH), jnp.float32),
    )(xp, W1, b1.reshape(1, -1), W2t, b2.reshape(-1, 1))


def kernel(diffusion_step, embedding, W1, b1, W2, b2):
    idx = diffusion_step.astype(jnp.int32)
    x = _sc_gather(embedding, idx)
    xp = x.reshape(_HALF, 2 * _D_IN)
    out_t = _pair_mlp_t(xp, W1, b1, W2.T, b2)
    return out_t.T
